# Initial kernel scaffold; baseline (speedup 1.0000x reference)
import jax
import jax.numpy as jnp
from jax.experimental import pallas as pl


def kernel(x):
    raise NotImplementedError("write your pallas kernel here")


# speedup vs baseline: 3.5063x; 3.5063x over previous
import jax
import jax.numpy as jnp
from jax.experimental import pallas as pl
from jax.experimental.pallas import tpu as pltpu

_B_BLK = 8


def _body(x_ref, o_ref):
    xb = x_ref[...]
    bits = jax.lax.bitcast_convert_type(xb, jnp.uint32)
    rows = jax.lax.broadcasted_iota(jnp.int32, xb.shape, dimension=1)
    cols = jax.lax.broadcasted_iota(jnp.int32, xb.shape, dimension=2)
    hit = jnp.logical_and(rows % 16 == 0, cols % 8 == 0)
    mask = jnp.where(hit, jnp.uint32(0xBFFFFFFF), jnp.uint32(0xFFFFFFFF))
    o_ref[...] = jax.lax.bitcast_convert_type(bits & mask, jnp.float32)


def kernel(x):
    n, r, c = x.shape
    grid = (n // _B_BLK,)
    return pl.pallas_call(
        _body,
        grid=grid,
        in_specs=[pl.BlockSpec((_B_BLK, r, c), lambda i: (i, 0, 0))],
        out_specs=pl.BlockSpec((_B_BLK, r, c), lambda i: (i, 0, 0)),
        out_shape=jax.ShapeDtypeStruct(x.shape, x.dtype),
    )(x)
